# Initial kernel scaffold; baseline (speedup 1.0000x reference)
import jax
import jax.numpy as jnp
from jax.experimental import pallas as pl


def kernel(rpn_cls_score, rpn_bbox_pred):
    raise NotImplementedError("write your pallas kernel here")


# speedup vs baseline: 18.8514x; 18.8514x over previous
import numpy as np
import jax
import jax.numpy as jnp
from jax.experimental import pallas as pl
from jax.experimental.pallas import tpu as pltpu

FEAT_STRIDE = 16
IM_DIMS = (512, 512)
ANCHOR_SCALES = (8, 16, 32)
ANCHOR_RATIOS = (0.5, 1.0, 2.0)
PRE_NMS = 2000
POST_NMS = 300
NMS_THRESH = 0.7
MIN_SIZE = 16.0

N_PAD = 2048
T = 128
K = N_PAD // T
OUT_PAD = 384


def _gen_base_anchors(base_size=16, ratios=ANCHOR_RATIOS, scales=ANCHOR_SCALES):
    base = np.array([0, 0, base_size - 1, base_size - 1], dtype=np.float64)

    def whctrs(a):
        w = a[2] - a[0] + 1.0
        h = a[3] - a[1] + 1.0
        return w, h, a[0] + 0.5 * (w - 1), a[1] + 0.5 * (h - 1)

    def mk(ws, hs, xc, yc):
        ws = np.asarray(ws, dtype=np.float64)[:, None]
        hs = np.asarray(hs, dtype=np.float64)[:, None]
        return np.hstack([xc - 0.5 * (ws - 1), yc - 0.5 * (hs - 1),
                          xc + 0.5 * (ws - 1), yc + 0.5 * (hs - 1)])

    w, h, xc, yc = whctrs(base)
    size = w * h
    sr = size / np.array(ratios)
    ws = np.round(np.sqrt(sr))
    hs = np.round(ws * np.array(ratios))
    ra = mk(ws, hs, xc, yc)
    out = []
    for i in range(ra.shape[0]):
        w, h, xc, yc = whctrs(ra[i])
        out.append(mk(w * np.array(scales), h * np.array(scales), xc, yc))
    return np.vstack(out).astype(np.float32)


def _grid_anchors(H, W):
    base = _gen_base_anchors()
    sy, sx = np.meshgrid(np.arange(H) * FEAT_STRIDE, np.arange(W) * FEAT_STRIDE,
                         indexing='ij')
    shifts = np.stack([sx.ravel(), sy.ravel(), sx.ravel(), sy.ravel()],
                      axis=1).astype(np.float32)
    return jnp.asarray((shifts[:, None, :] + base[None, :, :]).reshape(-1, 4))


def _decode(anchors, deltas):
    w = anchors[:, 2] - anchors[:, 0] + 1.0
    h = anchors[:, 3] - anchors[:, 1] + 1.0
    cx = anchors[:, 0] + 0.5 * w
    cy = anchors[:, 1] + 0.5 * h
    dx, dy, dw, dh = deltas[:, 0], deltas[:, 1], deltas[:, 2], deltas[:, 3]
    pcx = dx * w + cx
    pcy = dy * h + cy
    pw = jnp.exp(jnp.clip(dw, -10.0, 10.0)) * w
    ph = jnp.exp(jnp.clip(dh, -10.0, 10.0)) * h
    return jnp.stack([pcx - 0.5 * pw, pcy - 0.5 * ph,
                      pcx + 0.5 * pw, pcy + 0.5 * ph], axis=1)


def _nms_sel_kernel(sT_ref, cols_ref, out_ref, keep_ref, sub_ref, prev_ref):
    x1c = cols_ref[0:1, :]
    y1c = cols_ref[1:2, :]
    x2c = cols_ref[2:3, :]
    y2c = cols_ref[3:4, :]
    areac = (x2c - x1c + 1.0) * (y2c - y1c + 1.0)

    lane_n = jax.lax.broadcasted_iota(jnp.int32, (1, N_PAD), 1)
    lane_t = jax.lax.broadcasted_iota(jnp.int32, (1, T), 1)
    sub_t = jax.lax.broadcasted_iota(jnp.int32, (T, 1), 0)
    tri = (jax.lax.broadcasted_iota(jnp.int32, (T, T), 0) >
           jax.lax.broadcasted_iota(jnp.int32, (T, T), 1)).astype(jnp.float32)

    keep_ref[...] = jnp.zeros((1, N_PAD), jnp.float32)

    def block_step(k, carry):
        off = k * T
        bx1 = sT_ref[pl.ds(off, T), 0:1]
        by1 = sT_ref[pl.ds(off, T), 1:2]
        bx2 = sT_ref[pl.ds(off, T), 2:3]
        by2 = sT_ref[pl.ds(off, T), 3:4]
        bar = (bx2 - bx1 + 1.0) * (by2 - by1 + 1.0)
        xx1 = jnp.maximum(bx1, x1c)
        yy1 = jnp.maximum(by1, y1c)
        xx2 = jnp.minimum(bx2, x2c)
        yy2 = jnp.minimum(by2, y2c)
        iw = jnp.maximum(xx2 - xx1 + 1.0, 0.0)
        ih = jnp.maximum(yy2 - yy1 + 1.0, 0.0)
        inter = iw * ih
        iou = inter / (bar + areac - inter + 1e-9)
        sup = (iou > NMS_THRESH).astype(jnp.float32)
        keep = keep_ref[...]
        prev_ref[...] = jnp.sum(sup * keep, axis=1, keepdims=True)

        lx1 = cols_ref[0:1, pl.ds(off, T)]
        ly1 = cols_ref[1:2, pl.ds(off, T)]
        lx2 = cols_ref[2:3, pl.ds(off, T)]
        ly2 = cols_ref[3:4, pl.ds(off, T)]
        lar = (lx2 - lx1 + 1.0) * (ly2 - ly1 + 1.0)
        bxx1 = jnp.maximum(bx1, lx1)
        byy1 = jnp.maximum(by1, ly1)
        bxx2 = jnp.minimum(bx2, lx2)
        byy2 = jnp.minimum(by2, ly2)
        biw = jnp.maximum(bxx2 - bxx1 + 1.0, 0.0)
        bih = jnp.maximum(byy2 - byy1 + 1.0, 0.0)
        binter = biw * bih
        biou = binter / (bar + lar - binter + 1e-9)
        sub_ref[...] = (biou > NMS_THRESH).astype(jnp.float32) * tri

        def istep(i, kb):
            row = sub_ref[pl.ds(i, 1), :]
            p = prev_ref[pl.ds(i, 1), 0:1]
            s = p + jnp.sum(row * kb, axis=1, keepdims=True)
            v = (s == 0.0).astype(jnp.float32)
            return jnp.where(lane_t == i, v, kb)

        kb = jax.lax.fori_loop(0, T, istep, jnp.zeros((1, T), jnp.float32))
        keep_ref[0:1, pl.ds(off, T)] = kb
        return carry

    jax.lax.fori_loop(0, K, block_step, 0)
    keep = keep_ref[...]

    valid = (lane_n < PRE_NMS).astype(jnp.float32)
    kv = keep * valid
    nkv = (1.0 - keep) * valid
    upper = (jax.lax.broadcasted_iota(jnp.int32, (N_PAD, N_PAD), 0) <=
             jax.lax.broadcasted_iota(jnp.int32, (N_PAD, N_PAD), 1)
             ).astype(jnp.float32)
    both = jnp.concatenate([kv, nkv], axis=0)
    csums = jax.lax.dot_general(
        both, upper, (((1,), (0,)), ((), ())),
        preferred_element_type=jnp.float32)
    ck = csums[0:1, :]
    cn = csums[1:2, :]
    kcnt = ck[0:1, N_PAD - 1:N_PAD]
    pos = jnp.where(kv > 0, ck - 1.0,
                    jnp.where(nkv > 0, kcnt + cn - 1.0, 1e9))

    for t in range(OUT_PAD // T):
        svals = (t * T + sub_t).astype(jnp.float32)
        oh = (pos == svals).astype(jnp.float32)
        for c in range(4):
            col = cols_ref[c:c + 1, :]
            out_ref[pl.ds(t * T, T), c:c + 1] = jnp.sum(
                oh * col, axis=1, keepdims=True)


def _nms_select(boxes_sorted):
    cols = jnp.transpose(boxes_sorted)
    return pl.pallas_call(
        _nms_sel_kernel,
        out_shape=jax.ShapeDtypeStruct((OUT_PAD, 4), jnp.float32),
        scratch_shapes=[
            pltpu.VMEM((1, N_PAD), jnp.float32),
            pltpu.VMEM((T, T), jnp.float32),
            pltpu.VMEM((T, 1), jnp.float32),
        ],
    )(boxes_sorted, cols)


def kernel(rpn_cls_score, rpn_bbox_pred):
    B, H, W, c2 = rpn_cls_score.shape
    A = c2 // 2
    anchors = _grid_anchors(H, W)

    logits = rpn_cls_score.reshape(B, H, W, A, 2)
    probs = jax.nn.softmax(logits, axis=-1)
    scores = probs[..., 1].reshape(B, -1)
    deltas = rpn_bbox_pred.reshape(B, -1, 4)
    props = jax.vmap(lambda d: _decode(anchors, d))(deltas)
    im_h, im_w = IM_DIMS
    props = jnp.stack([
        jnp.clip(props[..., 0], 0.0, im_w - 1.0),
        jnp.clip(props[..., 1], 0.0, im_h - 1.0),
        jnp.clip(props[..., 2], 0.0, im_w - 1.0),
        jnp.clip(props[..., 3], 0.0, im_h - 1.0)], axis=-1)
    ws = props[..., 2] - props[..., 0] + 1.0
    hs = props[..., 3] - props[..., 1] + 1.0
    ok = (ws >= MIN_SIZE) & (hs >= MIN_SIZE)
    scores = jnp.where(ok, scores, -1e9)

    _, idx = jax.lax.top_k(scores, PRE_NMS)
    boxes = jnp.take_along_axis(props, idx[..., None], axis=1)
    boxes = jnp.pad(boxes, ((0, 0), (0, N_PAD - PRE_NMS), (0, 0)))

    sel = jax.vmap(_nms_select)(boxes)[:, :POST_NMS, :]
    bi = jnp.broadcast_to(
        jnp.arange(B, dtype=sel.dtype)[:, None, None], (B, POST_NMS, 1))
    return jnp.concatenate([bi, sel], axis=-1).reshape(B * POST_NMS, 5)


# speedup vs baseline: 86.7826x; 4.6035x over previous
import numpy as np
import jax
import jax.numpy as jnp
from jax.experimental import pallas as pl
from jax.experimental.pallas import tpu as pltpu

FEAT_STRIDE = 16
IM_DIMS = (512, 512)
ANCHOR_SCALES = (8, 16, 32)
ANCHOR_RATIOS = (0.5, 1.0, 2.0)
PRE_NMS = 2000
POST_NMS = 300
NMS_THRESH = 0.7
MIN_SIZE = 16.0

N_PAD = 2048
T = 128
K = N_PAD // T
OUT_PAD = 384


def _gen_base_anchors(base_size=16, ratios=ANCHOR_RATIOS, scales=ANCHOR_SCALES):
    base = np.array([0, 0, base_size - 1, base_size - 1], dtype=np.float64)

    def whctrs(a):
        w = a[2] - a[0] + 1.0
        h = a[3] - a[1] + 1.0
        return w, h, a[0] + 0.5 * (w - 1), a[1] + 0.5 * (h - 1)

    def mk(ws, hs, xc, yc):
        ws = np.asarray(ws, dtype=np.float64)[:, None]
        hs = np.asarray(hs, dtype=np.float64)[:, None]
        return np.hstack([xc - 0.5 * (ws - 1), yc - 0.5 * (hs - 1),
                          xc + 0.5 * (ws - 1), yc + 0.5 * (hs - 1)])

    w, h, xc, yc = whctrs(base)
    size = w * h
    sr = size / np.array(ratios)
    ws = np.round(np.sqrt(sr))
    hs = np.round(ws * np.array(ratios))
    ra = mk(ws, hs, xc, yc)
    out = []
    for i in range(ra.shape[0]):
        w, h, xc, yc = whctrs(ra[i])
        out.append(mk(w * np.array(scales), h * np.array(scales), xc, yc))
    return np.vstack(out).astype(np.float32)


def _grid_anchors(H, W):
    base = _gen_base_anchors()
    sy, sx = np.meshgrid(np.arange(H) * FEAT_STRIDE, np.arange(W) * FEAT_STRIDE,
                         indexing='ij')
    shifts = np.stack([sx.ravel(), sy.ravel(), sx.ravel(), sy.ravel()],
                      axis=1).astype(np.float32)
    return jnp.asarray((shifts[:, None, :] + base[None, :, :]).reshape(-1, 4))


def _decode(anchors, deltas):
    w = anchors[:, 2] - anchors[:, 0] + 1.0
    h = anchors[:, 3] - anchors[:, 1] + 1.0
    cx = anchors[:, 0] + 0.5 * w
    cy = anchors[:, 1] + 0.5 * h
    dx, dy, dw, dh = deltas[:, 0], deltas[:, 1], deltas[:, 2], deltas[:, 3]
    pcx = dx * w + cx
    pcy = dy * h + cy
    pw = jnp.exp(jnp.clip(dw, -10.0, 10.0)) * w
    ph = jnp.exp(jnp.clip(dh, -10.0, 10.0)) * h
    return jnp.stack([pcx - 0.5 * pw, pcy - 0.5 * ph,
                      pcx + 0.5 * pw, pcy + 0.5 * ph], axis=1)


def _nms_sel_kernel(sT_ref, cols_ref, out_ref, u_ref):
    x1c = cols_ref[0:1, :]
    y1c = cols_ref[1:2, :]
    x2c = cols_ref[2:3, :]
    y2c = cols_ref[3:4, :]
    areac = (x2c - x1c + 1.0) * (y2c - y1c + 1.0)

    lane_n = jax.lax.broadcasted_iota(jnp.int32, (1, N_PAD), 1)
    sub_t = jax.lax.broadcasted_iota(jnp.int32, (T, 1), 0)

    def build_step(k, carry):
        off = k * T
        bx1 = sT_ref[pl.ds(off, T), 0:1]
        by1 = sT_ref[pl.ds(off, T), 1:2]
        bx2 = sT_ref[pl.ds(off, T), 2:3]
        by2 = sT_ref[pl.ds(off, T), 3:4]
        bar = (bx2 - bx1 + 1.0) * (by2 - by1 + 1.0)
        xx1 = jnp.maximum(bx1, x1c)
        yy1 = jnp.maximum(by1, y1c)
        xx2 = jnp.minimum(bx2, x2c)
        yy2 = jnp.minimum(by2, y2c)
        iw = jnp.maximum(xx2 - xx1 + 1.0, 0.0)
        ih = jnp.maximum(yy2 - yy1 + 1.0, 0.0)
        inter = iw * ih
        iou = inter / (bar + areac - inter + 1e-9)
        order = ((off + sub_t) < lane_n).astype(jnp.float32)
        u_ref[pl.ds(off, T), :] = (iou > NMS_THRESH).astype(jnp.float32) * order
        return carry

    jax.lax.fori_loop(0, K, build_step, 0)
    u = u_ref[...]

    valid = (lane_n < PRE_NMS).astype(jnp.float32)

    def round_cond(state):
        c, _ = state
        return jnp.sum(c) > 0.0

    def round_body(state):
        c, kept = state
        supc = jax.lax.dot_general(
            c, u, (((1,), (0,)), ((), ())),
            preferred_element_type=jnp.float32)
        lead = c * (supc == 0.0).astype(jnp.float32)
        rej = jax.lax.dot_general(
            lead, u, (((1,), (0,)), ((), ())),
            preferred_element_type=jnp.float32)
        kept = kept + lead
        c = c * (1.0 - lead) * (rej == 0.0).astype(jnp.float32)
        return c, kept

    _, keep = jax.lax.while_loop(
        round_cond, round_body,
        (valid, jnp.zeros((1, N_PAD), jnp.float32)))

    kv = keep * valid
    nkv = (1.0 - keep) * valid
    upper = (jax.lax.broadcasted_iota(jnp.int32, (N_PAD, N_PAD), 0) <=
             jax.lax.broadcasted_iota(jnp.int32, (N_PAD, N_PAD), 1)
             ).astype(jnp.float32)
    both = jnp.concatenate([kv, nkv], axis=0)
    csums = jax.lax.dot_general(
        both, upper, (((1,), (0,)), ((), ())),
        preferred_element_type=jnp.float32)
    ck = csums[0:1, :]
    cn = csums[1:2, :]
    kcnt = ck[0:1, N_PAD - 1:N_PAD]
    pos = jnp.where(kv > 0, ck - 1.0,
                    jnp.where(nkv > 0, kcnt + cn - 1.0, 1e9))

    for t in range(OUT_PAD // T):
        svals = (t * T + sub_t).astype(jnp.float32)
        oh = (pos == svals).astype(jnp.float32)
        for c in range(4):
            col = cols_ref[c:c + 1, :]
            out_ref[pl.ds(t * T, T), c:c + 1] = jnp.sum(
                oh * col, axis=1, keepdims=True)


def _nms_select(boxes_sorted):
    cols = jnp.transpose(boxes_sorted)
    return pl.pallas_call(
        _nms_sel_kernel,
        out_shape=jax.ShapeDtypeStruct((OUT_PAD, 4), jnp.float32),
        scratch_shapes=[
            pltpu.VMEM((N_PAD, N_PAD), jnp.float32),
        ],
    )(boxes_sorted, cols)


def kernel(rpn_cls_score, rpn_bbox_pred):
    B, H, W, c2 = rpn_cls_score.shape
    A = c2 // 2
    anchors = _grid_anchors(H, W)

    logits = rpn_cls_score.reshape(B, H, W, A, 2)
    probs = jax.nn.softmax(logits, axis=-1)
    scores = probs[..., 1].reshape(B, -1)
    deltas = rpn_bbox_pred.reshape(B, -1, 4)
    props = jax.vmap(lambda d: _decode(anchors, d))(deltas)
    im_h, im_w = IM_DIMS
    props = jnp.stack([
        jnp.clip(props[..., 0], 0.0, im_w - 1.0),
        jnp.clip(props[..., 1], 0.0, im_h - 1.0),
        jnp.clip(props[..., 2], 0.0, im_w - 1.0),
        jnp.clip(props[..., 3], 0.0, im_h - 1.0)], axis=-1)
    ws = props[..., 2] - props[..., 0] + 1.0
    hs = props[..., 3] - props[..., 1] + 1.0
    ok = (ws >= MIN_SIZE) & (hs >= MIN_SIZE)
    scores = jnp.where(ok, scores, -1e9)

    _, idx = jax.lax.top_k(scores, PRE_NMS)
    boxes = jnp.take_along_axis(props, idx[..., None], axis=1)
    boxes = jnp.pad(boxes, ((0, 0), (0, N_PAD - PRE_NMS), (0, 0)))

    sel = jax.vmap(_nms_select)(boxes)[:, :POST_NMS, :]
    bi = jnp.broadcast_to(
        jnp.arange(B, dtype=sel.dtype)[:, None, None], (B, POST_NMS, 1))
    return jnp.concatenate([bi, sel], axis=-1).reshape(B * POST_NMS, 5)
